# baseline (speedup 1.0000x reference)
import jax
import jax.numpy as jnp
from jax.experimental import pallas as pl
from jax.experimental.pallas import tpu as pltpu

_B, _S, _BULK, _TOWER = 8, 2048, 512, 128
_NUM_LEVELS = 16
_LN_EPS = 1e-6
_BLK = 2048


def _fold8(s):
    a = s[0:4, :] + s[4:8, :]
    b = a[0:2, :] + a[2:4, :]
    return b[0:1, :] + b[1:2, :]


def _sum128_t(a):
    s = a[0:8, :]
    for t in range(1, 16):
        s = s + a[8 * t:8 * t + 8, :]
    return _fold8(s)


def _d8_nearest_t(y):
    r = jnp.round(y)
    err = y - r
    a = jnp.abs(err)
    m = jnp.max(a, axis=0, keepdims=True)
    ii = jax.lax.broadcasted_iota(jnp.int32, y.shape, 0)
    cand = jnp.where(a >= m, ii, 8)
    sel = jnp.min(cand, axis=0, keepdims=True)
    onehot = (ii == sel).astype(y.dtype)
    sign = jnp.where(err >= 0.0, 1.0, -1.0)
    odd = _fold8(r).astype(jnp.int32) & 1
    return r + sign * onehot * odd.astype(y.dtype)


def _e8_nearest_t(x):
    a = _d8_nearest_t(x)
    b = _d8_nearest_t(x - 0.5) + 0.5
    xa = x - a
    xb = x - b
    da = _fold8(xa * xa)
    db = _fold8(xb * xb)
    return jnp.where(da <= db, a, b)


def _layernorm_rows(h, g, b):
    mu = jnp.mean(h, axis=-1, keepdims=True)
    var = jnp.mean((h - mu) ** 2, axis=-1, keepdims=True)
    return (h - mu) * jax.lax.rsqrt(var + _LN_EPS) * g + b


def _block_kernel(
    x_ref, w_bt, b_bt, g_et, be_et, w_te, b_te, rms_s,
    w_et, b_et, g_dt, be_dt, w_tb, b_tb, g_db, be_db,
    out_ref, codes_ref,
):
    hb = x_ref[...]

    mm1t = jax.lax.dot_general(
        w_bt[...], hb, (((0,), (1,)), ((), ())),
        preferred_element_type=jnp.float32,
    ) + b_bt[...]
    mu = _sum128_t(mm1t) * (1.0 / 128.0)
    d = mm1t - mu
    var = _sum128_t(d * d) * (1.0 / 128.0)
    towert = d * (1.0 / jnp.sqrt(var + _LN_EPS)) * g_et[...] + be_et[...]
    e8t = jax.lax.dot_general(
        w_te[...], towert, (((0,), (0,)), ((), ())),
        preferred_element_type=jnp.float32,
    ) + b_te[...]
    ms8 = _fold8(e8t * e8t) * (1.0 / 8.0)
    u = e8t * (1.0 / jnp.sqrt(ms8 + _LN_EPS)) * rms_s[...]

    quant = jnp.zeros_like(u)
    code_rows = []
    scale = 1.0
    for _ in range(_NUM_LEVELS):
        lat = _e8_nearest_t(u)
        code_rows.append((2.0 * lat).astype(jnp.int32))
        quant = quant + scale * lat
        u = 2.0 * (u - lat)
        scale = scale * 0.5

    codes_t = jnp.concatenate(code_rows, axis=0)
    codes_ref[...] = codes_t.T

    dt = jax.lax.dot_general(
        quant, w_et[...], (((0,), (0,)), ((), ())),
        preferred_element_type=jnp.float32,
    ) + b_et[...]
    dt = _layernorm_rows(dt, g_dt[...], be_dt[...])
    bulk = jnp.dot(dt, w_tb[...], preferred_element_type=jnp.float32) + b_tb[...]
    out_ref[...] = _layernorm_rows(bulk, g_db[...], be_db[...])


@jax.jit
def _run(x, params):
    n = _B * _S
    h = x.reshape(n, _BULK)

    def row2d(v):
        return v.reshape(1, -1)

    def col2d(v):
        return v.reshape(-1, 1)

    w_args = (
        params['W_bt'], col2d(params['b_bt']),
        col2d(params['g_enc_t']), col2d(params['be_enc_t']),
        params['W_te'], col2d(params['b_te']),
        col2d(params['rms_scale']),
        params['W_et'], row2d(params['b_et']),
        row2d(params['g_dec_t']), row2d(params['be_dec_t']),
        params['W_tb'], row2d(params['b_tb']),
        row2d(params['g_dec_b']), row2d(params['be_dec_b']),
    )

    def full(a):
        return pl.BlockSpec(a.shape, lambda i: (0,) * a.ndim)

    in_specs = [pl.BlockSpec((_BLK, _BULK), lambda i: (i, 0))]
    in_specs += [full(a) for a in w_args]

    out, codes = pl.pallas_call(
        _block_kernel,
        grid=(n // _BLK,),
        in_specs=in_specs,
        out_specs=[
            pl.BlockSpec((_BLK, _BULK), lambda i: (i, 0)),
            pl.BlockSpec((_BLK, _NUM_LEVELS * 8), lambda i: (i, 0)),
        ],
        out_shape=[
            jax.ShapeDtypeStruct((n, _BULK), jnp.float32),
            jax.ShapeDtypeStruct((n, _NUM_LEVELS * 8), jnp.int32),
        ],
        compiler_params=pltpu.CompilerParams(
            dimension_semantics=("parallel",),
        ),
    )(h, *w_args)

    recon = out.reshape(_B, _S, _BULK)
    codes = codes.reshape(_B, _S, _NUM_LEVELS, 8)
    return recon, codes


def kernel(x, params):
    return _run(x, params)


# speedup vs baseline: 1.0047x; 1.0047x over previous
import jax
import jax.numpy as jnp
from jax.experimental import pallas as pl
from jax.experimental.pallas import tpu as pltpu

_B, _S, _BULK, _TOWER = 8, 2048, 512, 128
_NUM_LEVELS = 16
_LN_EPS = 1e-6
_BLK = 4096


def _fold8(s):
    a = s[0:4, :] + s[4:8, :]
    b = a[0:2, :] + a[2:4, :]
    return b[0:1, :] + b[1:2, :]


def _sum128_t(a):
    s = a[0:8, :]
    for t in range(1, 16):
        s = s + a[8 * t:8 * t + 8, :]
    return _fold8(s)


def _d8_nearest_t(y):
    r = jnp.round(y)
    err = y - r
    a = jnp.abs(err)
    m = jnp.max(a, axis=0, keepdims=True)
    ii = jax.lax.broadcasted_iota(jnp.int32, y.shape, 0)
    cand = jnp.where(a >= m, ii, 8)
    sel = jnp.min(cand, axis=0, keepdims=True)
    onehot = (ii == sel).astype(y.dtype)
    sign = jnp.where(err >= 0.0, 1.0, -1.0)
    odd = _fold8(r).astype(jnp.int32) & 1
    return r + sign * onehot * odd.astype(y.dtype)


def _e8_nearest_t(x):
    a = _d8_nearest_t(x)
    b = _d8_nearest_t(x - 0.5) + 0.5
    xa = x - a
    xb = x - b
    da = _fold8(xa * xa)
    db = _fold8(xb * xb)
    return jnp.where(da <= db, a, b)


def _layernorm_rows(h, g, b):
    mu = jnp.mean(h, axis=-1, keepdims=True)
    var = jnp.mean((h - mu) ** 2, axis=-1, keepdims=True)
    return (h - mu) * jax.lax.rsqrt(var + _LN_EPS) * g + b


def _block_kernel(
    x_ref, w_bt, b_bt, g_et, be_et, w_te, b_te, rms_s,
    w_et, b_et, g_dt, be_dt, w_tb, b_tb, g_db, be_db,
    out_ref, codes_ref,
):
    hb = x_ref[...]

    mm1t = jax.lax.dot_general(
        w_bt[...], hb, (((0,), (1,)), ((), ())),
        preferred_element_type=jnp.float32,
    ) + b_bt[...]
    mu = _sum128_t(mm1t) * (1.0 / 128.0)
    d = mm1t - mu
    var = _sum128_t(d * d) * (1.0 / 128.0)
    towert = d * (1.0 / jnp.sqrt(var + _LN_EPS)) * g_et[...] + be_et[...]
    e8t = jax.lax.dot_general(
        w_te[...], towert, (((0,), (0,)), ((), ())),
        preferred_element_type=jnp.float32,
    ) + b_te[...]
    ms8 = _fold8(e8t * e8t) * (1.0 / 8.0)
    u = e8t * (1.0 / jnp.sqrt(ms8 + _LN_EPS)) * rms_s[...]

    quant = jnp.zeros_like(u)
    code_rows = []
    scale = 1.0
    for _ in range(_NUM_LEVELS):
        lat = _e8_nearest_t(u)
        code_rows.append((2.0 * lat).astype(jnp.int32))
        quant = quant + scale * lat
        u = 2.0 * (u - lat)
        scale = scale * 0.5

    codes_t = jnp.concatenate(code_rows, axis=0)
    codes_ref[...] = codes_t.T

    dt = jax.lax.dot_general(
        quant, w_et[...], (((0,), (0,)), ((), ())),
        preferred_element_type=jnp.float32,
    ) + b_et[...]
    dt = _layernorm_rows(dt, g_dt[...], be_dt[...])
    bulk = jnp.dot(dt, w_tb[...], preferred_element_type=jnp.float32) + b_tb[...]
    out_ref[...] = _layernorm_rows(bulk, g_db[...], be_db[...])


@jax.jit
def _run(x, params):
    n = _B * _S
    h = x.reshape(n, _BULK)

    def row2d(v):
        return v.reshape(1, -1)

    def col2d(v):
        return v.reshape(-1, 1)

    w_args = (
        params['W_bt'], col2d(params['b_bt']),
        col2d(params['g_enc_t']), col2d(params['be_enc_t']),
        params['W_te'], col2d(params['b_te']),
        col2d(params['rms_scale']),
        params['W_et'], row2d(params['b_et']),
        row2d(params['g_dec_t']), row2d(params['be_dec_t']),
        params['W_tb'], row2d(params['b_tb']),
        row2d(params['g_dec_b']), row2d(params['be_dec_b']),
    )

    def full(a):
        return pl.BlockSpec(a.shape, lambda i: (0,) * a.ndim)

    in_specs = [pl.BlockSpec((_BLK, _BULK), lambda i: (i, 0))]
    in_specs += [full(a) for a in w_args]

    out, codes = pl.pallas_call(
        _block_kernel,
        grid=(n // _BLK,),
        in_specs=in_specs,
        out_specs=[
            pl.BlockSpec((_BLK, _BULK), lambda i: (i, 0)),
            pl.BlockSpec((_BLK, _NUM_LEVELS * 8), lambda i: (i, 0)),
        ],
        out_shape=[
            jax.ShapeDtypeStruct((n, _BULK), jnp.float32),
            jax.ShapeDtypeStruct((n, _NUM_LEVELS * 8), jnp.int32),
        ],
        compiler_params=pltpu.CompilerParams(
            dimension_semantics=("parallel",),
        ),
    )(h, *w_args)

    recon = out.reshape(_B, _S, _BULK)
    codes = codes.reshape(_B, _S, _NUM_LEVELS, 8)
    return recon, codes


def kernel(x, params):
    return _run(x, params)
